# baseline (speedup 1.0000x reference)
import jax
import jax.numpy as jnp
from jax import lax
from jax.experimental import pallas as pl
from jax.experimental.pallas import tpu as pltpu
from jax.experimental.pallas import tpu_sc as plsc

_NC = 2
_NS = 16
_NW = _NC * _NS

_BATCH = 16384
_EMB = 16
_LANES = 128
_PACK = _LANES // _EMB


def _sc_gather_body(r_sess, r_item, t_sg, t_ig, t_sm, t_im,
                    o_sg, o_ig, o_sm, o_im,
                    idxs, idxi, buf, sem):
  bpw = _BATCH // _NW
  wid = lax.axis_index("s") * _NC + lax.axis_index("c")
  base = wid * bpw
  pltpu.sync_copy(r_sess.at[pl.ds(base, bpw)], idxs)
  pltpu.sync_copy(r_item.at[pl.ds(base, bpw)], idxi)
  for tab, idx, out in ((t_sg, idxs, o_sg), (t_ig, idxi, o_ig),
                        (t_sm, idxs, o_sm), (t_im, idxi, o_im)):
    pltpu.async_copy(tab.at[idx], buf, sem).wait()
    pltpu.sync_copy(buf, out.at[pl.ds(base, bpw)])


def _sc_gather(r_sess, r_item, t_sg, t_ig, t_sm, t_im):
  bpw = _BATCH // _NW
  mesh = plsc.VectorSubcoreMesh(core_axis_name="c", subcore_axis_name="s",
                                num_cores=_NC, num_subcores=_NS)
  row = jax.ShapeDtypeStruct((_BATCH, _LANES), jnp.float32)
  f = pl.kernel(
      _sc_gather_body,
      out_type=[row, row, row, row],
      mesh=mesh,
      scratch_types=[
          pltpu.VMEM((bpw,), jnp.int32),
          pltpu.VMEM((bpw,), jnp.int32),
          pltpu.VMEM((bpw, _LANES), jnp.float32),
          pltpu.SemaphoreType.DMA,
      ],
  )
  return f(r_sess, r_item, t_sg, t_ig, t_sm, t_im)


def _extract(x128, off):
  acc = jnp.zeros((x128.shape[0], _EMB), jnp.float32)
  for k in range(_PACK):
    sel = x128[:, _EMB * k:_EMB * (k + 1)]
    acc += jnp.where(off == _EMB * k, sel, 0.0)
  return acc


def _tc_mlp_body(sg1, ig1, sm1, im1, cs, ci, w1, b1, w2, b2, w3, b3, wo, bo,
                 out):
  cs_v = cs[...]
  ci_v = ci[...]
  sm = _extract(sm1[...], cs_v)
  im = _extract(im1[...], ci_v)
  dn = (((1,), (1,)), ((), ()))
  w1v = w1[...]
  h = lax.dot_general(sm, w1v[:, :_EMB], dn,
                      preferred_element_type=jnp.float32)
  h += lax.dot_general(im, w1v[:, _EMB:], dn,
                       preferred_element_type=jnp.float32)
  h = jnp.maximum(h + b1[...], 0.0)
  h = lax.dot_general(h, w2[...], dn, preferred_element_type=jnp.float32)
  h = jnp.maximum(h + b2[...], 0.0)
  h = lax.dot_general(h, w3[...], dn, preferred_element_type=jnp.float32)
  h = jnp.maximum(h + b3[...], 0.0)
  gmf = _extract(sg1[...], cs_v) * _extract(ig1[...], ci_v)
  wov = wo[...]
  logit = lax.dot_general(gmf, wov[:, :_EMB], dn,
                          preferred_element_type=jnp.float32)
  logit += lax.dot_general(h, wov[:, _EMB:], dn,
                           preferred_element_type=jnp.float32)
  out[...] = jax.nn.sigmoid(logit + bo[...])


def _tc_mlp(sg1, ig1, sm1, im1, cs, ci, w1, b1, w2, b2, w3, b3, wo, bo):
  chunk = 2048
  grid = (_BATCH // chunk,)
  big_spec = pl.BlockSpec((chunk, _LANES), lambda i: (i, 0))
  col_spec = pl.BlockSpec((chunk, 1), lambda i: (i, 0))

  def full(shape):
    return pl.BlockSpec(shape, lambda i: tuple(0 for _ in shape))

  out = pl.pallas_call(
      _tc_mlp_body,
      grid=grid,
      in_specs=[
          big_spec, big_spec, big_spec, big_spec,
          col_spec, col_spec,
          full((64, 32)), full((64,)),
          full((32, 64)), full((32,)),
          full((16, 32)), full((16,)),
          full((1, 32)), full((1,)),
      ],
      out_specs=pl.BlockSpec((chunk, 1), lambda i: (i, 0)),
      out_shape=jax.ShapeDtypeStruct((_BATCH, 1), jnp.float32),
  )(sg1, ig1, sm1, im1, cs, ci, w1, b1, w2, b2, w3, b3, wo, bo)
  return jnp.squeeze(out, axis=-1)


@jax.jit
def kernel(sess_ids, item_ids, sess_emb_gmf, item_emb_gmf, sess_emb_mlp,
           item_emb_mlp, W1, b1, W2, b2, W3, b3, Wo, bo):
  n_rows = sess_emb_gmf.shape[0] // _PACK
  t_sg = jnp.reshape(sess_emb_gmf, (n_rows, _LANES))
  t_ig = jnp.reshape(item_emb_gmf, (n_rows, _LANES))
  t_sm = jnp.reshape(sess_emb_mlp, (n_rows, _LANES))
  t_im = jnp.reshape(item_emb_mlp, (n_rows, _LANES))
  r_sess = lax.shift_right_logical(sess_ids, 3)
  r_item = lax.shift_right_logical(item_ids, 3)
  cs = jnp.reshape((sess_ids & 7) * _EMB, (_BATCH, 1))
  ci = jnp.reshape((item_ids & 7) * _EMB, (_BATCH, 1))
  sg1, ig1, sm1, im1 = _sc_gather(r_sess, r_item, t_sg, t_ig, t_sm, t_im)
  return _tc_mlp(sg1, ig1, sm1, im1, cs, ci, W1, b1, W2, b2, W3, b3, Wo, bo)


# speedup vs baseline: 1.0646x; 1.0646x over previous
import jax
import jax.numpy as jnp
from jax import lax
from jax.experimental import pallas as pl
from jax.experimental.pallas import tpu as pltpu
from jax.experimental.pallas import tpu_sc as plsc

_NC = 2
_NS = 16
_NW = _NC * _NS

_BATCH = 16384
_EMB = 16
_BPW = _BATCH // _NW
_CH = 128
_NCH = _BPW // _CH


def _sc_gather_body(sess_ids, item_ids, t_sg, t_ig, t_sm, t_im,
                    o_sg, o_ig, o_sm, o_im,
                    idxs, idxi, bsg, big, bsm, bim,
                    sem0, sem1, sem2, sem3):
  wid = lax.axis_index("s") * _NC + lax.axis_index("c")
  base = wid * _BPW
  for k in range(_NCH):
    pltpu.sync_copy(sess_ids.at[pl.ds(base + k * _CH, _CH)], idxs.at[k])
    pltpu.sync_copy(item_ids.at[pl.ds(base + k * _CH, _CH)], idxi.at[k])

  handles = []
  for k in range(_NCH):
    dst = pl.ds(k * _CH, _CH)
    handles.append(pltpu.async_copy(t_sg.at[idxs.at[k]], bsg.at[dst], sem0))
    handles.append(pltpu.async_copy(t_ig.at[idxi.at[k]], big.at[dst], sem1))
    handles.append(pltpu.async_copy(t_sm.at[idxs.at[k]], bsm.at[dst], sem2))
    handles.append(pltpu.async_copy(t_im.at[idxi.at[k]], bim.at[dst], sem3))
  for h in handles:
    h.wait()

  for buf, out in ((bsg, o_sg), (big, o_ig), (bsm, o_sm), (bim, o_im)):
    pltpu.sync_copy(buf, out.at[pl.ds(base, _BPW)])


def _sc_gather(sess_ids, item_ids, t_sg, t_ig, t_sm, t_im):
  mesh = plsc.VectorSubcoreMesh(core_axis_name="c", subcore_axis_name="s",
                                num_cores=_NC, num_subcores=_NS)
  rows = jax.ShapeDtypeStruct((_BATCH, _EMB), jnp.float32)
  buf = pltpu.VMEM((_BPW, _EMB), jnp.float32)
  f = pl.kernel(
      _sc_gather_body,
      out_type=[rows, rows, rows, rows],
      mesh=mesh,
      scratch_types=[
          pltpu.VMEM((_NCH, _CH), jnp.int32),
          pltpu.VMEM((_NCH, _CH), jnp.int32),
          buf, buf, buf, buf,
          pltpu.SemaphoreType.DMA,
          pltpu.SemaphoreType.DMA,
          pltpu.SemaphoreType.DMA,
          pltpu.SemaphoreType.DMA,
      ],
      compiler_params=pltpu.CompilerParams(use_tc_tiling_on_sc=False),
  )
  return f(sess_ids, item_ids, t_sg, t_ig, t_sm, t_im)


def _tc_mlp_body(sg, ig, sm, im, w1, b1, w2, b2, w3, b3, wo, bo, out):
  dn = (((1,), (1,)), ((), ()))
  w1v = w1[...]
  h = lax.dot_general(sm[...], w1v[:, :_EMB], dn,
                      preferred_element_type=jnp.float32)
  h += lax.dot_general(im[...], w1v[:, _EMB:], dn,
                       preferred_element_type=jnp.float32)
  h = jnp.maximum(h + b1[...], 0.0)
  h = lax.dot_general(h, w2[...], dn, preferred_element_type=jnp.float32)
  h = jnp.maximum(h + b2[...], 0.0)
  h = lax.dot_general(h, w3[...], dn, preferred_element_type=jnp.float32)
  h = jnp.maximum(h + b3[...], 0.0)
  gmf = sg[...] * ig[...]
  wov = wo[...]
  logit = lax.dot_general(gmf, wov[:, :_EMB], dn,
                          preferred_element_type=jnp.float32)
  logit += lax.dot_general(h, wov[:, _EMB:], dn,
                           preferred_element_type=jnp.float32)
  out[...] = jax.nn.sigmoid(logit + bo[...])


def _tc_mlp(sg, ig, sm, im, w1, b1, w2, b2, w3, b3, wo, bo):
  chunk = 4096
  grid = (_BATCH // chunk,)
  act_spec = pl.BlockSpec((chunk, _EMB), lambda i: (i, 0))

  def full(shape):
    return pl.BlockSpec(shape, lambda i: tuple(0 for _ in shape))

  out = pl.pallas_call(
      _tc_mlp_body,
      grid=grid,
      in_specs=[
          act_spec, act_spec, act_spec, act_spec,
          full((64, 32)), full((1, 64)),
          full((32, 64)), full((1, 32)),
          full((16, 32)), full((1, 16)),
          full((1, 32)), full((1, 1)),
      ],
      out_specs=pl.BlockSpec((chunk, 1), lambda i: (i, 0)),
      out_shape=jax.ShapeDtypeStruct((_BATCH, 1), jnp.float32),
  )(sg, ig, sm, im, w1, b1, w2, b2, w3, b3, wo, bo)
  return jnp.reshape(out, (_BATCH,))


@jax.jit
def kernel(sess_ids, item_ids, sess_emb_gmf, item_emb_gmf, sess_emb_mlp,
           item_emb_mlp, W1, b1, W2, b2, W3, b3, Wo, bo):
  sg, ig, sm, im = _sc_gather(sess_ids, item_ids, sess_emb_gmf,
                              item_emb_gmf, sess_emb_mlp, item_emb_mlp)
  return _tc_mlp(sg, ig, sm, im, W1,
                 jnp.reshape(b1, (1, 64)), W2, jnp.reshape(b2, (1, 32)),
                 W3, jnp.reshape(b3, (1, 16)), Wo, jnp.reshape(bo, (1, 1)))


# speedup vs baseline: 1.3323x; 1.2514x over previous
import jax
import jax.numpy as jnp
from jax import lax
from jax.experimental import pallas as pl
from jax.experimental.pallas import tpu as pltpu
from jax.experimental.pallas import tpu_sc as plsc

_NC = 2
_NS = 16
_NW = _NC * _NS

_BATCH = 16384
_EMB = 16
_BPW = _BATCH // _NW
_CH = 128
_NCH = _BPW // _CH

_PACK = 128 // _EMB
_PROWS = 1000000 // _PACK
_RBLK = 1024


def _repack_body(t0, t1, t2, t3, o0, o1, o2, o3, xt):
  eye = (lax.broadcasted_iota(jnp.int32, (_EMB, _EMB), 0)
         == lax.broadcasted_iota(jnp.int32, (_EMB, _EMB), 1)
         ).astype(jnp.float32)
  dn = (((0,), (0,)), ((), ()))
  for t, o in ((t0, o0), (t1, o1), (t2, o2), (t3, o3)):
    xt[...] = lax.dot_general(t[...], eye, dn,
                              preferred_element_type=jnp.float32)
    for j in range(_PACK):
      o[:, j * _EMB:(j + 1) * _EMB] = xt[j::_PACK, :]


def _tc_repack(t_sg, t_ig, t_sm, t_im):
  grid = (pl.cdiv(_PROWS, _RBLK),)
  in_spec = pl.BlockSpec((_EMB, _PACK * _RBLK), lambda i: (0, i))
  out_spec = pl.BlockSpec((_RBLK, 128), lambda i: (i, 0))
  packed = jax.ShapeDtypeStruct((_PROWS, 128), jnp.float32)
  return pl.pallas_call(
      _repack_body,
      grid=grid,
      in_specs=[in_spec] * 4,
      out_specs=[out_spec] * 4,
      out_shape=[packed] * 4,
      scratch_shapes=[pltpu.VMEM((_PACK * _RBLK, _EMB), jnp.float32)],
  )(t_sg, t_ig, t_sm, t_im)


def _sc_gather_body(row_w, sidx, iidx, t_sg, t_ig, t_sm, t_im,
                    o_sg, o_ig, o_sm, o_im,
                    idxs, idxi, bsg, big, bsm, bim,
                    sem0, sem1, sem2, sem3):
  wid = lax.axis_index("s") * _NC + lax.axis_index("c")
  base = wid * _BPW
  for k in range(_NCH):
    pltpu.sync_copy(sidx.at[pl.ds(base + k * _CH, _CH)], idxs.at[k])
    pltpu.sync_copy(iidx.at[pl.ds(base + k * _CH, _CH)], idxi.at[k])

  for k in range(_NCH):
    h0 = pltpu.async_copy(t_sg.at[idxs.at[k]], bsg, sem0)
    h1 = pltpu.async_copy(t_ig.at[idxi.at[k]], big, sem1)
    h2 = pltpu.async_copy(t_sm.at[idxs.at[k]], bsm, sem2)
    h3 = pltpu.async_copy(t_im.at[idxi.at[k]], bim, sem3)
    h0.wait()
    h1.wait()
    h2.wait()
    h3.wait()
    dst = pl.ds(base + k * _CH, _CH)
    for buf, out in ((bsg, o_sg), (big, o_ig), (bsm, o_sm), (bim, o_im)):
      pltpu.sync_copy(buf, out.at[dst])


def _sc_gather(row_w, sidx, iidx, t_sg, t_ig, t_sm, t_im):
  mesh = plsc.VectorSubcoreMesh(core_axis_name="c", subcore_axis_name="s",
                                num_cores=_NC, num_subcores=_NS)
  rows = jax.ShapeDtypeStruct((_BATCH, row_w), jnp.float32)
  buf = pltpu.VMEM((_CH, row_w), jnp.float32)
  f = pl.kernel(
      lambda *a: _sc_gather_body(row_w, *a),
      out_type=[rows, rows, rows, rows],
      mesh=mesh,
      scratch_types=[
          pltpu.VMEM((_NCH, _CH), jnp.int32),
          pltpu.VMEM((_NCH, _CH), jnp.int32),
          buf, buf, buf, buf,
          pltpu.SemaphoreType.DMA,
          pltpu.SemaphoreType.DMA,
          pltpu.SemaphoreType.DMA,
          pltpu.SemaphoreType.DMA,
      ],
      compiler_params=pltpu.CompilerParams(use_tc_tiling_on_sc=False),
  )
  return f(sidx, iidx, t_sg, t_ig, t_sm, t_im)


def _tc_mlp_body(sg, ig, sm, im, w1, b1, w2, b2, w3, b3, wo, bo, out):
  dn = (((1,), (1,)), ((), ()))
  w1v = w1[...]
  h = lax.dot_general(sm[...], w1v[:, :_EMB], dn,
                      preferred_element_type=jnp.float32)
  h += lax.dot_general(im[...], w1v[:, _EMB:], dn,
                       preferred_element_type=jnp.float32)
  h = jnp.maximum(h + b1[...], 0.0)
  h = lax.dot_general(h, w2[...], dn, preferred_element_type=jnp.float32)
  h = jnp.maximum(h + b2[...], 0.0)
  h = lax.dot_general(h, w3[...], dn, preferred_element_type=jnp.float32)
  h = jnp.maximum(h + b3[...], 0.0)
  gmf = sg[...] * ig[...]
  wov = wo[...]
  logit = lax.dot_general(gmf, wov[:, :_EMB], dn,
                          preferred_element_type=jnp.float32)
  logit += lax.dot_general(h, wov[:, _EMB:], dn,
                           preferred_element_type=jnp.float32)
  out[...] = jax.nn.sigmoid(logit + bo[...])


def _tc_mlp(sg, ig, sm, im, w1, b1, w2, b2, w3, b3, wo, bo):
  chunk = 4096
  grid = (_BATCH // chunk,)
  act_spec = pl.BlockSpec((chunk, _EMB), lambda i: (i, 0))

  def full(shape):
    return pl.BlockSpec(shape, lambda i: tuple(0 for _ in shape))

  out = pl.pallas_call(
      _tc_mlp_body,
      grid=grid,
      in_specs=[
          act_spec, act_spec, act_spec, act_spec,
          full((64, 32)), full((1, 64)),
          full((32, 64)), full((1, 32)),
          full((16, 32)), full((1, 16)),
          full((1, 32)), full((1, 1)),
      ],
      out_specs=pl.BlockSpec((chunk, 1), lambda i: (i, 0)),
      out_shape=jax.ShapeDtypeStruct((_BATCH, 1), jnp.float32),
  )(sg, ig, sm, im, w1, b1, w2, b2, w3, b3, wo, bo)
  return jnp.reshape(out, (_BATCH,))


@jax.jit
def kernel(sess_ids, item_ids, sess_emb_gmf, item_emb_gmf, sess_emb_mlp,
           item_emb_mlp, W1, b1, W2, b2, W3, b3, Wo, bo):
  rp_sg, rp_ig, rp_sm, rp_im = _tc_repack(
      sess_emb_gmf.T, item_emb_gmf.T, sess_emb_mlp.T, item_emb_mlp.T)

  sidx1 = jax.lax.shift_right_logical(sess_ids, 3)
  iidx1 = jax.lax.shift_right_logical(item_ids, 3)
  rows_sg, rows_ig, rows_sm, rows_im = _sc_gather(
      128, sidx1, iidx1, rp_sg, rp_ig, rp_sm, rp_im)

  r = jnp.arange(_BATCH, dtype=jnp.int32) * _PACK
  sidx2 = r + jnp.bitwise_and(sess_ids, _PACK - 1)
  iidx2 = r + jnp.bitwise_and(item_ids, _PACK - 1)
  flat = lambda x: jnp.reshape(x, (_BATCH * _PACK, _EMB))
  sg, ig, sm, im = _sc_gather(
      _EMB, sidx2, iidx2, flat(rows_sg), flat(rows_ig),
      flat(rows_sm), flat(rows_im))

  return _tc_mlp(sg, ig, sm, im, W1,
                 jnp.reshape(b1, (1, 64)), W2, jnp.reshape(b2, (1, 32)),
                 W3, jnp.reshape(b3, (1, 16)), Wo, jnp.reshape(bo, (1, 1)))


# speedup vs baseline: 1.4280x; 1.0719x over previous
import jax
import jax.numpy as jnp
from jax import lax
from jax.experimental import pallas as pl
from jax.experimental.pallas import tpu as pltpu
from jax.experimental.pallas import tpu_sc as plsc

_NC = 2
_NS = 16
_NW = _NC * _NS

_BATCH = 16384
_EMB = 16
_BPW = _BATCH // _NW
_CH = 128
_NCH = _BPW // _CH

_PACK = 128 // _EMB
_PROWS = 1000000 // _PACK
_RBLK = 1024


def _repack_body(t0, t1, t2, t3, o0, o1, o2, o3, xt):
  eye = (lax.broadcasted_iota(jnp.int32, (_EMB, _EMB), 0)
         == lax.broadcasted_iota(jnp.int32, (_EMB, _EMB), 1)
         ).astype(jnp.float32)
  dn = (((0,), (0,)), ((), ()))
  for t, o in ((t0, o0), (t1, o1), (t2, o2), (t3, o3)):
    xt[...] = lax.dot_general(t[...], eye, dn,
                              preferred_element_type=jnp.float32)
    for q in range(_PACK):
      o[:, q * _EMB:(q + 1) * _EMB] = xt[q * _RBLK:(q + 1) * _RBLK, :]


def _tc_repack(t_sg, t_ig, t_sm, t_im):
  grid = (pl.cdiv(_PROWS, _RBLK),)
  in_spec = pl.BlockSpec((_EMB, _PACK * _RBLK), lambda i: (0, i))
  out_spec = pl.BlockSpec((_RBLK, 128), lambda i: (i, 0))
  packed = jax.ShapeDtypeStruct((pl.cdiv(_PROWS, _RBLK) * _RBLK, 128),
                                jnp.float32)
  return pl.pallas_call(
      _repack_body,
      grid=grid,
      in_specs=[in_spec] * 4,
      out_specs=[out_spec] * 4,
      out_shape=[packed] * 4,
      scratch_shapes=[pltpu.VMEM((_PACK * _RBLK, _EMB), jnp.float32)],
  )(t_sg, t_ig, t_sm, t_im)


def _sc_gather_body(row_w, sidx, iidx, t_sg, t_ig, t_sm, t_im,
                    o_sg, o_ig, o_sm, o_im,
                    idxs, idxi, bsg, big, bsm, bim,
                    sem0, sem1, sem2, sem3):
  wid = lax.axis_index("s") * _NC + lax.axis_index("c")
  base = wid * _BPW
  for k in range(_NCH):
    pltpu.sync_copy(sidx.at[pl.ds(base + k * _CH, _CH)], idxs.at[k])
    pltpu.sync_copy(iidx.at[pl.ds(base + k * _CH, _CH)], idxi.at[k])

  for k in range(_NCH):
    h0 = pltpu.async_copy(t_sg.at[idxs.at[k]], bsg, sem0)
    h1 = pltpu.async_copy(t_ig.at[idxi.at[k]], big, sem1)
    h2 = pltpu.async_copy(t_sm.at[idxs.at[k]], bsm, sem2)
    h3 = pltpu.async_copy(t_im.at[idxi.at[k]], bim, sem3)
    h0.wait()
    h1.wait()
    h2.wait()
    h3.wait()
    dst = pl.ds(base + k * _CH, _CH)
    for buf, out in ((bsg, o_sg), (big, o_ig), (bsm, o_sm), (bim, o_im)):
      pltpu.sync_copy(buf, out.at[dst])


def _sc_gather(row_w, sidx, iidx, t_sg, t_ig, t_sm, t_im):
  mesh = plsc.VectorSubcoreMesh(core_axis_name="c", subcore_axis_name="s",
                                num_cores=_NC, num_subcores=_NS)
  rows = jax.ShapeDtypeStruct((_BATCH, row_w), jnp.float32)
  buf = pltpu.VMEM((_CH, row_w), jnp.float32)
  f = pl.kernel(
      lambda *a: _sc_gather_body(row_w, *a),
      out_type=[rows, rows, rows, rows],
      mesh=mesh,
      scratch_types=[
          pltpu.VMEM((_NCH, _CH), jnp.int32),
          pltpu.VMEM((_NCH, _CH), jnp.int32),
          buf, buf, buf, buf,
          pltpu.SemaphoreType.DMA,
          pltpu.SemaphoreType.DMA,
          pltpu.SemaphoreType.DMA,
          pltpu.SemaphoreType.DMA,
      ],
      compiler_params=pltpu.CompilerParams(use_tc_tiling_on_sc=False),
  )
  return f(sidx, iidx, t_sg, t_ig, t_sm, t_im)


def _tc_mlp_body(sg, ig, sm, im, w1, b1, w2, b2, w3, b3, wo, bo, out):
  dn = (((1,), (1,)), ((), ()))
  w1v = w1[...]
  h = lax.dot_general(sm[...], w1v[:, :_EMB], dn,
                      preferred_element_type=jnp.float32)
  h += lax.dot_general(im[...], w1v[:, _EMB:], dn,
                       preferred_element_type=jnp.float32)
  h = jnp.maximum(h + b1[...], 0.0)
  h = lax.dot_general(h, w2[...], dn, preferred_element_type=jnp.float32)
  h = jnp.maximum(h + b2[...], 0.0)
  h = lax.dot_general(h, w3[...], dn, preferred_element_type=jnp.float32)
  h = jnp.maximum(h + b3[...], 0.0)
  gmf = sg[...] * ig[...]
  wov = wo[...]
  logit = lax.dot_general(gmf, wov[:, :_EMB], dn,
                          preferred_element_type=jnp.float32)
  logit += lax.dot_general(h, wov[:, _EMB:], dn,
                           preferred_element_type=jnp.float32)
  out[...] = jax.nn.sigmoid(logit + bo[...])


def _tc_mlp(sg, ig, sm, im, w1, b1, w2, b2, w3, b3, wo, bo):
  chunk = 4096
  grid = (_BATCH // chunk,)
  act_spec = pl.BlockSpec((chunk, _EMB), lambda i: (i, 0))

  def full(shape):
    return pl.BlockSpec(shape, lambda i: tuple(0 for _ in shape))

  out = pl.pallas_call(
      _tc_mlp_body,
      grid=grid,
      in_specs=[
          act_spec, act_spec, act_spec, act_spec,
          full((64, 32)), full((1, 64)),
          full((32, 64)), full((1, 32)),
          full((16, 32)), full((1, 16)),
          full((1, 32)), full((1, 1)),
      ],
      out_specs=pl.BlockSpec((chunk, 1), lambda i: (i, 0)),
      out_shape=jax.ShapeDtypeStruct((_BATCH, 1), jnp.float32),
  )(sg, ig, sm, im, w1, b1, w2, b2, w3, b3, wo, bo)
  return jnp.reshape(out, (_BATCH,))


@jax.jit
def kernel(sess_ids, item_ids, sess_emb_gmf, item_emb_gmf, sess_emb_mlp,
           item_emb_mlp, W1, b1, W2, b2, W3, b3, Wo, bo):
  rp_sg, rp_ig, rp_sm, rp_im = _tc_repack(
      sess_emb_gmf.T, item_emb_gmf.T, sess_emb_mlp.T, item_emb_mlp.T)

  def row1(ids):
    return (jax.lax.shift_right_logical(ids, 13) * _RBLK
            + jnp.bitwise_and(ids, _RBLK - 1))

  rows_sg, rows_ig, rows_sm, rows_im = _sc_gather(
      128, row1(sess_ids), row1(item_ids), rp_sg, rp_ig, rp_sm, rp_im)

  r = jnp.arange(_BATCH, dtype=jnp.int32) * _PACK

  def row2(ids):
    return r + jnp.bitwise_and(jax.lax.shift_right_logical(ids, 10),
                               _PACK - 1)

  sidx2 = row2(sess_ids)
  iidx2 = row2(item_ids)
  flat = lambda x: jnp.reshape(x, (_BATCH * _PACK, _EMB))
  sg, ig, sm, im = _sc_gather(
      _EMB, sidx2, iidx2, flat(rows_sg), flat(rows_ig),
      flat(rows_sm), flat(rows_im))

  return _tc_mlp(sg, ig, sm, im, W1,
                 jnp.reshape(b1, (1, 64)), W2, jnp.reshape(b2, (1, 32)),
                 W3, jnp.reshape(b3, (1, 16)), Wo, jnp.reshape(bo, (1, 1)))


# speedup vs baseline: 2.4504x; 1.7159x over previous
import jax
import jax.numpy as jnp
from jax import lax
from jax.experimental import pallas as pl
from jax.experimental.pallas import tpu as pltpu
from jax.experimental.pallas import tpu_sc as plsc

_NC = 2
_NS = 16
_NW = _NC * _NS

_BATCH = 16384
_EMB = 16
_BPW = _BATCH // _NW
_CH = 128
_NCH = _BPW // _CH

_PACK = 128 // _EMB
_PROWS = 1000000 // _PACK
_RBLK = 1024


def _repack_body(t0, t1, t2, t3, o0, o1, o2, o3):
  f_row = lax.broadcasted_iota(jnp.int32, (_EMB, 128), 0)
  w_col = lax.broadcasted_iota(jnp.int32, (_EMB, 128), 1)
  dn = (((0,), (0,)), ((), ()))
  for t, o in ((t0, o0), (t1, o1), (t2, o2), (t3, o3)):
    tv = t[...]
    acc = None
    for q in range(_PACK):
      eq = (w_col == q * _EMB + f_row).astype(jnp.float32)
      y = lax.dot_general(tv[:, q * _RBLK:(q + 1) * _RBLK], eq, dn,
                          preferred_element_type=jnp.float32)
      acc = y if acc is None else acc + y
    o[...] = acc


def _tc_repack(t_sg, t_ig, t_sm, t_im):
  grid = (pl.cdiv(_PROWS, _RBLK),)
  in_spec = pl.BlockSpec((_EMB, _PACK * _RBLK), lambda i: (0, i))
  out_spec = pl.BlockSpec((_RBLK, 128), lambda i: (i, 0))
  packed = jax.ShapeDtypeStruct((pl.cdiv(_PROWS, _RBLK) * _RBLK, 128),
                                jnp.float32)
  return pl.pallas_call(
      _repack_body,
      grid=grid,
      in_specs=[in_spec] * 4,
      out_specs=[out_spec] * 4,
      out_shape=[packed] * 4,
  )(t_sg, t_ig, t_sm, t_im)


def _sc_gather_body(row_w, sidx, iidx, t_sg, t_ig, t_sm, t_im,
                    o_sg, o_ig, o_sm, o_im,
                    idxs, idxi, bsg, big, bsm, bim,
                    sem0, sem1, sem2, sem3):
  wid = lax.axis_index("s") * _NC + lax.axis_index("c")
  base = wid * _BPW
  for k in range(_NCH):
    pltpu.sync_copy(sidx.at[pl.ds(base + k * _CH, _CH)], idxs.at[k])
    pltpu.sync_copy(iidx.at[pl.ds(base + k * _CH, _CH)], idxi.at[k])

  for k in range(_NCH):
    h0 = pltpu.async_copy(t_sg.at[idxs.at[k]], bsg, sem0)
    h1 = pltpu.async_copy(t_ig.at[idxi.at[k]], big, sem1)
    h2 = pltpu.async_copy(t_sm.at[idxs.at[k]], bsm, sem2)
    h3 = pltpu.async_copy(t_im.at[idxi.at[k]], bim, sem3)
    h0.wait()
    h1.wait()
    h2.wait()
    h3.wait()
    dst = pl.ds(base + k * _CH, _CH)
    for buf, out in ((bsg, o_sg), (big, o_ig), (bsm, o_sm), (bim, o_im)):
      pltpu.sync_copy(buf, out.at[dst])


def _sc_gather(row_w, sidx, iidx, t_sg, t_ig, t_sm, t_im):
  mesh = plsc.VectorSubcoreMesh(core_axis_name="c", subcore_axis_name="s",
                                num_cores=_NC, num_subcores=_NS)
  rows = jax.ShapeDtypeStruct((_BATCH, row_w), jnp.float32)
  buf = pltpu.VMEM((_CH, row_w), jnp.float32)
  f = pl.kernel(
      lambda *a: _sc_gather_body(row_w, *a),
      out_type=[rows, rows, rows, rows],
      mesh=mesh,
      scratch_types=[
          pltpu.VMEM((_NCH, _CH), jnp.int32),
          pltpu.VMEM((_NCH, _CH), jnp.int32),
          buf, buf, buf, buf,
          pltpu.SemaphoreType.DMA,
          pltpu.SemaphoreType.DMA,
          pltpu.SemaphoreType.DMA,
          pltpu.SemaphoreType.DMA,
      ],
      compiler_params=pltpu.CompilerParams(use_tc_tiling_on_sc=False),
  )
  return f(sidx, iidx, t_sg, t_ig, t_sm, t_im)


def _tc_mlp_body(sg, ig, sm, im, w1, b1, w2, b2, w3, b3, wo, bo, out):
  dn = (((1,), (1,)), ((), ()))
  w1v = w1[...]
  h = lax.dot_general(sm[...], w1v[:, :_EMB], dn,
                      preferred_element_type=jnp.float32)
  h += lax.dot_general(im[...], w1v[:, _EMB:], dn,
                       preferred_element_type=jnp.float32)
  h = jnp.maximum(h + b1[...], 0.0)
  h = lax.dot_general(h, w2[...], dn, preferred_element_type=jnp.float32)
  h = jnp.maximum(h + b2[...], 0.0)
  h = lax.dot_general(h, w3[...], dn, preferred_element_type=jnp.float32)
  h = jnp.maximum(h + b3[...], 0.0)
  gmf = sg[...] * ig[...]
  wov = wo[...]
  logit = lax.dot_general(gmf, wov[:, :_EMB], dn,
                          preferred_element_type=jnp.float32)
  logit += lax.dot_general(h, wov[:, _EMB:], dn,
                           preferred_element_type=jnp.float32)
  out[...] = jax.nn.sigmoid(logit + bo[...])


def _tc_mlp(sg, ig, sm, im, w1, b1, w2, b2, w3, b3, wo, bo):
  chunk = 4096
  grid = (_BATCH // chunk,)
  act_spec = pl.BlockSpec((chunk, _EMB), lambda i: (i, 0))

  def full(shape):
    return pl.BlockSpec(shape, lambda i: tuple(0 for _ in shape))

  out = pl.pallas_call(
      _tc_mlp_body,
      grid=grid,
      in_specs=[
          act_spec, act_spec, act_spec, act_spec,
          full((64, 32)), full((1, 64)),
          full((32, 64)), full((1, 32)),
          full((16, 32)), full((1, 16)),
          full((1, 32)), full((1, 1)),
      ],
      out_specs=pl.BlockSpec((chunk, 1), lambda i: (i, 0)),
      out_shape=jax.ShapeDtypeStruct((_BATCH, 1), jnp.float32),
  )(sg, ig, sm, im, w1, b1, w2, b2, w3, b3, wo, bo)
  return jnp.reshape(out, (_BATCH,))


@jax.jit
def kernel(sess_ids, item_ids, sess_emb_gmf, item_emb_gmf, sess_emb_mlp,
           item_emb_mlp, W1, b1, W2, b2, W3, b3, Wo, bo):
  rp_sg, rp_ig, rp_sm, rp_im = _tc_repack(
      sess_emb_gmf.T, item_emb_gmf.T, sess_emb_mlp.T, item_emb_mlp.T)

  def row1(ids):
    return (jax.lax.shift_right_logical(ids, 13) * _RBLK
            + jnp.bitwise_and(ids, _RBLK - 1))

  rows_sg, rows_ig, rows_sm, rows_im = _sc_gather(
      128, row1(sess_ids), row1(item_ids), rp_sg, rp_ig, rp_sm, rp_im)

  r = jnp.arange(_BATCH, dtype=jnp.int32) * _PACK

  def row2(ids):
    return r + jnp.bitwise_and(jax.lax.shift_right_logical(ids, 10),
                               _PACK - 1)

  sidx2 = row2(sess_ids)
  iidx2 = row2(item_ids)
  flat = lambda x: jnp.reshape(x, (_BATCH * _PACK, _EMB))
  sg, ig, sm, im = _sc_gather(
      _EMB, sidx2, iidx2, flat(rows_sg), flat(rows_ig),
      flat(rows_sm), flat(rows_im))

  return _tc_mlp(sg, ig, sm, im, W1,
                 jnp.reshape(b1, (1, 64)), W2, jnp.reshape(b2, (1, 32)),
                 W3, jnp.reshape(b3, (1, 16)), Wo, jnp.reshape(bo, (1, 1)))


# speedup vs baseline: 2.4981x; 1.0195x over previous
import jax
import jax.numpy as jnp
from jax import lax
from jax.experimental import pallas as pl
from jax.experimental.pallas import tpu as pltpu
from jax.experimental.pallas import tpu_sc as plsc

_NC = 2
_NS = 16
_NW = _NC * _NS

_BATCH = 16384
_EMB = 16
_BPW = _BATCH // _NW
_CH = 128
_NCH = _BPW // _CH

_PACK = 128 // _EMB
_PROWS = 1000000 // _PACK
_RBLK = 4096
_RSH = _RBLK.bit_length() - 1


def _repack_body(t0, t1, t2, t3, o0, o1, o2, o3):
  f_row = lax.broadcasted_iota(jnp.int32, (_EMB, 128), 0)
  w_col = lax.broadcasted_iota(jnp.int32, (_EMB, 128), 1)
  dn = (((0,), (0,)), ((), ()))
  for t, o in ((t0, o0), (t1, o1), (t2, o2), (t3, o3)):
    tv = t[...]
    acc = None
    for q in range(_PACK):
      eq = (w_col == q * _EMB + f_row).astype(jnp.float32)
      y = lax.dot_general(tv[:, q * _RBLK:(q + 1) * _RBLK], eq, dn,
                          preferred_element_type=jnp.float32)
      acc = y if acc is None else acc + y
    o[...] = acc


def _tc_repack(t_sg, t_ig, t_sm, t_im):
  grid = (pl.cdiv(_PROWS, _RBLK),)
  in_spec = pl.BlockSpec((_EMB, _PACK * _RBLK), lambda i: (0, i))
  out_spec = pl.BlockSpec((_RBLK, 128), lambda i: (i, 0))
  packed = jax.ShapeDtypeStruct((pl.cdiv(_PROWS, _RBLK) * _RBLK, 128),
                                jnp.float32)
  return pl.pallas_call(
      _repack_body,
      grid=grid,
      in_specs=[in_spec] * 4,
      out_specs=[out_spec] * 4,
      out_shape=[packed] * 4,
  )(t_sg, t_ig, t_sm, t_im)


def _sc_gather_body(row_w, sidx, iidx, t_sg, t_ig, t_sm, t_im,
                    o_sg, o_ig, o_sm, o_im,
                    idxs, idxi, bsg, big, bsm, bim,
                    sem0, sem1, sem2, sem3):
  wid = lax.axis_index("s") * _NC + lax.axis_index("c")
  base = wid * _BPW
  for k in range(_NCH):
    pltpu.sync_copy(sidx.at[pl.ds(base + k * _CH, _CH)], idxs.at[k])
    pltpu.sync_copy(iidx.at[pl.ds(base + k * _CH, _CH)], idxi.at[k])

  for k in range(_NCH):
    h0 = pltpu.async_copy(t_sg.at[idxs.at[k]], bsg, sem0)
    h1 = pltpu.async_copy(t_ig.at[idxi.at[k]], big, sem1)
    h2 = pltpu.async_copy(t_sm.at[idxs.at[k]], bsm, sem2)
    h3 = pltpu.async_copy(t_im.at[idxi.at[k]], bim, sem3)
    h0.wait()
    h1.wait()
    h2.wait()
    h3.wait()
    dst = pl.ds(base + k * _CH, _CH)
    for buf, out in ((bsg, o_sg), (big, o_ig), (bsm, o_sm), (bim, o_im)):
      pltpu.sync_copy(buf, out.at[dst])


def _sc_gather(row_w, sidx, iidx, t_sg, t_ig, t_sm, t_im):
  mesh = plsc.VectorSubcoreMesh(core_axis_name="c", subcore_axis_name="s",
                                num_cores=_NC, num_subcores=_NS)
  rows = jax.ShapeDtypeStruct((_BATCH, row_w), jnp.float32)
  buf = pltpu.VMEM((_CH, row_w), jnp.float32)
  f = pl.kernel(
      lambda *a: _sc_gather_body(row_w, *a),
      out_type=[rows, rows, rows, rows],
      mesh=mesh,
      scratch_types=[
          pltpu.VMEM((_NCH, _CH), jnp.int32),
          pltpu.VMEM((_NCH, _CH), jnp.int32),
          buf, buf, buf, buf,
          pltpu.SemaphoreType.DMA,
          pltpu.SemaphoreType.DMA,
          pltpu.SemaphoreType.DMA,
          pltpu.SemaphoreType.DMA,
      ],
      compiler_params=pltpu.CompilerParams(use_tc_tiling_on_sc=False),
  )
  return f(sidx, iidx, t_sg, t_ig, t_sm, t_im)


def _tc_mlp_body(sg, ig, sm, im, w1, b1, w2, b2, w3, b3, wo, bo, out):
  dn = (((1,), (1,)), ((), ()))
  w1v = w1[...]
  h = lax.dot_general(sm[...], w1v[:, :_EMB], dn,
                      preferred_element_type=jnp.float32)
  h += lax.dot_general(im[...], w1v[:, _EMB:], dn,
                       preferred_element_type=jnp.float32)
  h = jnp.maximum(h + b1[...], 0.0)
  h = lax.dot_general(h, w2[...], dn, preferred_element_type=jnp.float32)
  h = jnp.maximum(h + b2[...], 0.0)
  h = lax.dot_general(h, w3[...], dn, preferred_element_type=jnp.float32)
  h = jnp.maximum(h + b3[...], 0.0)
  gmf = sg[...] * ig[...]
  wov = wo[...]
  logit = lax.dot_general(gmf, wov[:, :_EMB], dn,
                          preferred_element_type=jnp.float32)
  logit += lax.dot_general(h, wov[:, _EMB:], dn,
                           preferred_element_type=jnp.float32)
  out[...] = jax.nn.sigmoid(logit + bo[...])


def _tc_mlp(sg, ig, sm, im, w1, b1, w2, b2, w3, b3, wo, bo):
  chunk = 4096
  grid = (_BATCH // chunk,)
  act_spec = pl.BlockSpec((chunk, _EMB), lambda i: (i, 0))

  def full(shape):
    return pl.BlockSpec(shape, lambda i: tuple(0 for _ in shape))

  out = pl.pallas_call(
      _tc_mlp_body,
      grid=grid,
      in_specs=[
          act_spec, act_spec, act_spec, act_spec,
          full((64, 32)), full((1, 64)),
          full((32, 64)), full((1, 32)),
          full((16, 32)), full((1, 16)),
          full((1, 32)), full((1, 1)),
      ],
      out_specs=pl.BlockSpec((chunk, 1), lambda i: (i, 0)),
      out_shape=jax.ShapeDtypeStruct((_BATCH, 1), jnp.float32),
  )(sg, ig, sm, im, w1, b1, w2, b2, w3, b3, wo, bo)
  return jnp.reshape(out, (_BATCH,))


@jax.jit
def kernel(sess_ids, item_ids, sess_emb_gmf, item_emb_gmf, sess_emb_mlp,
           item_emb_mlp, W1, b1, W2, b2, W3, b3, Wo, bo):
  rp_sg, rp_ig, rp_sm, rp_im = _tc_repack(
      sess_emb_gmf.T, item_emb_gmf.T, sess_emb_mlp.T, item_emb_mlp.T)

  def row1(ids):
    return (jax.lax.shift_right_logical(ids, _RSH + 3) * _RBLK
            + jnp.bitwise_and(ids, _RBLK - 1))

  rows_sg, rows_ig, rows_sm, rows_im = _sc_gather(
      128, row1(sess_ids), row1(item_ids), rp_sg, rp_ig, rp_sm, rp_im)

  r = jnp.arange(_BATCH, dtype=jnp.int32) * _PACK

  def row2(ids):
    return r + jnp.bitwise_and(jax.lax.shift_right_logical(ids, _RSH),
                               _PACK - 1)

  sidx2 = row2(sess_ids)
  iidx2 = row2(item_ids)
  flat = lambda x: jnp.reshape(x, (_BATCH * _PACK, _EMB))
  sg, ig, sm, im = _sc_gather(
      _EMB, sidx2, iidx2, flat(rows_sg), flat(rows_ig),
      flat(rows_sm), flat(rows_im))

  return _tc_mlp(sg, ig, sm, im, W1,
                 jnp.reshape(b1, (1, 64)), W2, jnp.reshape(b2, (1, 32)),
                 W3, jnp.reshape(b3, (1, 16)), Wo, jnp.reshape(bo, (1, 1)))


# speedup vs baseline: 2.5860x; 1.0352x over previous
import jax
import jax.numpy as jnp
from jax import lax
from jax.experimental import pallas as pl
from jax.experimental.pallas import tpu as pltpu
from jax.experimental.pallas import tpu_sc as plsc

_NC = 2
_NS = 16
_NW = _NC * _NS

_BATCH = 16384
_EMB = 16
_BPW = _BATCH // _NW
_CH = 128
_NCH = _BPW // _CH

_PACK = 128 // _EMB
_PROWS = 1000000 // _PACK
_RBLK = 4096
_RSH = _RBLK.bit_length() - 1


def _repack_body(t0, t1, t2, t3, o0, o1, o2, o3):
  f_row = lax.broadcasted_iota(jnp.int32, (_EMB, 128), 0)
  w_col = lax.broadcasted_iota(jnp.int32, (_EMB, 128), 1)
  dn = (((0,), (0,)), ((), ()))
  for t, o in ((t0, o0), (t1, o1), (t2, o2), (t3, o3)):
    tv = t[...]
    acc = None
    for q in range(_PACK):
      eq = (w_col == q * _EMB + f_row).astype(jnp.float32)
      y = lax.dot_general(tv[:, q * _RBLK:(q + 1) * _RBLK], eq, dn,
                          preferred_element_type=jnp.float32)
      acc = y if acc is None else acc + y
    o[...] = acc


def _tc_repack(t_sg, t_ig, t_sm, t_im):
  grid = (pl.cdiv(_PROWS, _RBLK),)
  in_spec = pl.BlockSpec((_EMB, _PACK * _RBLK), lambda i: (0, i))
  out_spec = pl.BlockSpec((_RBLK, 128), lambda i: (i, 0))
  packed = jax.ShapeDtypeStruct((pl.cdiv(_PROWS, _RBLK) * _RBLK, 128),
                                jnp.float32)
  return pl.pallas_call(
      _repack_body,
      grid=grid,
      in_specs=[in_spec] * 4,
      out_specs=[out_spec] * 4,
      out_shape=[packed] * 4,
  )(t_sg, t_ig, t_sm, t_im)


def _sc_gather_body(row_w, sidx, iidx, t_sg, t_ig, t_sm, t_im,
                    o_sg, o_ig, o_sm, o_im,
                    idxs, idxi, bsg, big, bsm, bim,
                    sem0, sem1, sem2, sem3):
  wid = lax.axis_index("s") * _NC + lax.axis_index("c")
  base = wid * _BPW
  for k in range(_NCH):
    pltpu.sync_copy(sidx.at[pl.ds(base + k * _CH, _CH)], idxs.at[k])
    pltpu.sync_copy(iidx.at[pl.ds(base + k * _CH, _CH)], idxi.at[k])

  for k in range(_NCH):
    h0 = pltpu.async_copy(t_sg.at[idxs.at[k]], bsg, sem0)
    h1 = pltpu.async_copy(t_ig.at[idxi.at[k]], big, sem1)
    h2 = pltpu.async_copy(t_sm.at[idxs.at[k]], bsm, sem2)
    h3 = pltpu.async_copy(t_im.at[idxi.at[k]], bim, sem3)
    h0.wait()
    h1.wait()
    h2.wait()
    h3.wait()
    dst = pl.ds(base + k * _CH, _CH)
    for buf, out in ((bsg, o_sg), (big, o_ig), (bsm, o_sm), (bim, o_im)):
      pltpu.sync_copy(buf, out.at[dst])


def _sc_gather(row_w, sidx, iidx, t_sg, t_ig, t_sm, t_im):
  mesh = plsc.VectorSubcoreMesh(core_axis_name="c", subcore_axis_name="s",
                                num_cores=_NC, num_subcores=_NS)
  rows = jax.ShapeDtypeStruct((_BATCH, row_w), jnp.float32)
  buf = pltpu.VMEM((_CH, row_w), jnp.float32)
  f = pl.kernel(
      lambda *a: _sc_gather_body(row_w, *a),
      out_type=[rows, rows, rows, rows],
      mesh=mesh,
      scratch_types=[
          pltpu.VMEM((_NCH, _CH), jnp.int32),
          pltpu.VMEM((_NCH, _CH), jnp.int32),
          buf, buf, buf, buf,
          pltpu.SemaphoreType.DMA,
          pltpu.SemaphoreType.DMA,
          pltpu.SemaphoreType.DMA,
          pltpu.SemaphoreType.DMA,
      ],
      compiler_params=pltpu.CompilerParams(use_tc_tiling_on_sc=False),
  )
  return f(sidx, iidx, t_sg, t_ig, t_sm, t_im)


_G = _BATCH // _PACK


def _tc_mlp_body(sg, ig, sm, im, w1s, w1i, b1, w2, b2, w3, b3, wog, woh,
                 bo, out, h_ref):
  dnp = (((1,), (0,)), ((), ()))
  dn = (((1,), (1,)), ((), ()))
  smv = sm[...]
  imv = im[...]
  for q in range(_PACK):
    h = lax.dot_general(smv, w1s[q], dnp, preferred_element_type=jnp.float32)
    h += lax.dot_general(imv, w1i[q], dnp, preferred_element_type=jnp.float32)
    h_ref[q * _G:(q + 1) * _G, :] = jnp.maximum(h + b1[...], 0.0)
  h = lax.dot_general(h_ref[...], w2[...], dn,
                      preferred_element_type=jnp.float32)
  h = jnp.maximum(h + b2[...], 0.0)
  h = lax.dot_general(h, w3[...], dn, preferred_element_type=jnp.float32)
  h = jnp.maximum(h + b3[...], 0.0)
  gmf = sg[...] * ig[...]
  for q in range(_PACK):
    logit = lax.dot_general(gmf, wog[q], dnp,
                            preferred_element_type=jnp.float32)
    logit += lax.dot_general(h[q * _G:(q + 1) * _G, :], woh[...], dn,
                             preferred_element_type=jnp.float32)
    out[q * _G:(q + 1) * _G, :] = jax.nn.sigmoid(logit + bo[...])


def _tc_mlp(sg, ig, sm, im, W1, b1, W2, b2, W3, b3, Wo, bo):
  w = jnp.arange(128)
  f = jnp.arange(_EMB)
  q = jnp.arange(_PACK)
  E = (w[None, :, None] == (q[:, None, None] * _EMB + f[None, None, :])
       ).astype(jnp.float32)
  w1s = jnp.einsum("qwf,of->qwo", E, W1[:, :_EMB])
  w1i = jnp.einsum("qwf,of->qwo", E, W1[:, _EMB:])
  wog = jnp.einsum("qwf,of->qwo", E, Wo[:, :_EMB])
  woh = Wo[:, _EMB:]

  def full(shape):
    return pl.BlockSpec(shape, lambda: tuple(0 for _ in shape))

  pk = lambda x: jnp.reshape(x, (_G, 128))
  out = pl.pallas_call(
      _tc_mlp_body,
      in_specs=[
          full((_G, 128)), full((_G, 128)), full((_G, 128)), full((_G, 128)),
          full((_PACK, 128, 64)), full((_PACK, 128, 64)), full((1, 64)),
          full((32, 64)), full((1, 32)),
          full((16, 32)), full((1, 16)),
          full((_PACK, 128, 1)), full((1, _EMB)), full((1, 1)),
      ],
      out_specs=full((_BATCH, 1)),
      out_shape=jax.ShapeDtypeStruct((_BATCH, 1), jnp.float32),
      scratch_shapes=[pltpu.VMEM((_BATCH, 64), jnp.float32)],
  )(pk(sg), pk(ig), pk(sm), pk(im), w1s, w1i,
    jnp.reshape(b1, (1, 64)), W2, jnp.reshape(b2, (1, 32)),
    W3, jnp.reshape(b3, (1, 16)), wog, woh, jnp.reshape(bo, (1, 1)))
  return jnp.reshape(jnp.transpose(jnp.reshape(out, (_PACK, _G))), (_BATCH,))


@jax.jit
def kernel(sess_ids, item_ids, sess_emb_gmf, item_emb_gmf, sess_emb_mlp,
           item_emb_mlp, W1, b1, W2, b2, W3, b3, Wo, bo):
  rp_sg, rp_ig, rp_sm, rp_im = _tc_repack(
      sess_emb_gmf.T, item_emb_gmf.T, sess_emb_mlp.T, item_emb_mlp.T)

  def row1(ids):
    return (jax.lax.shift_right_logical(ids, _RSH + 3) * _RBLK
            + jnp.bitwise_and(ids, _RBLK - 1))

  rows_sg, rows_ig, rows_sm, rows_im = _sc_gather(
      128, row1(sess_ids), row1(item_ids), rp_sg, rp_ig, rp_sm, rp_im)

  r = jnp.arange(_BATCH, dtype=jnp.int32) * _PACK

  def row2(ids):
    return r + jnp.bitwise_and(jax.lax.shift_right_logical(ids, _RSH),
                               _PACK - 1)

  sidx2 = row2(sess_ids)
  iidx2 = row2(item_ids)
  flat = lambda x: jnp.reshape(x, (_BATCH * _PACK, _EMB))
  sg, ig, sm, im = _sc_gather(
      _EMB, sidx2, iidx2, flat(rows_sg), flat(rows_ig),
      flat(rows_sm), flat(rows_im))

  return _tc_mlp(sg, ig, sm, im, W1, b1, W2, b2, W3, b3, Wo, bo)


# speedup vs baseline: 4.4377x; 1.7160x over previous
import jax
import jax.numpy as jnp
from jax import lax
from jax.experimental import pallas as pl
from jax.experimental.pallas import tpu as pltpu
from jax.experimental.pallas import tpu_sc as plsc

_NC = 2
_NS = 16
_NW = _NC * _NS

_BATCH = 16384
_EMB = 16
_BPW = _BATCH // _NW
_CH = 128
_NCH = _BPW // _CH

_PACK = 128 // _EMB
_PROWS = 1000000 // _PACK
_RBLK = 4096
_RSH = _RBLK.bit_length() - 1


def _repack_body(t0, t1, t2, t3, o0, o1, o2, o3):
  x4 = jnp.concatenate([t0[...], t1[...], t2[...], t3[...]], axis=0)
  j_row = lax.broadcasted_iota(jnp.int32, (4 * _EMB, 512), 0)
  l_col = lax.broadcasted_iota(jnp.int32, (4 * _EMB, 512), 1)
  tgt = (j_row // _EMB) * 128 + (j_row % _EMB)
  dn = (((0,), (0,)), ((), ()))
  acc = None
  for q in range(_PACK):
    eq = (l_col == tgt + q * _EMB).astype(jnp.float32)
    y = lax.dot_general(x4[:, q * _RBLK:(q + 1) * _RBLK], eq, dn,
                        preferred_element_type=jnp.float32)
    acc = y if acc is None else acc + y
  for t_i, o in enumerate((o0, o1, o2, o3)):
    o[...] = acc[:, t_i * 128:(t_i + 1) * 128]


def _tc_repack(t_sg, t_ig, t_sm, t_im):
  grid = (pl.cdiv(_PROWS, _RBLK),)
  in_spec = pl.BlockSpec((_EMB, _PACK * _RBLK), lambda i: (0, i))
  out_spec = pl.BlockSpec((_RBLK, 128), lambda i: (i, 0))
  packed = jax.ShapeDtypeStruct((pl.cdiv(_PROWS, _RBLK) * _RBLK, 128),
                                jnp.float32)
  return pl.pallas_call(
      _repack_body,
      grid=grid,
      in_specs=[in_spec] * 4,
      out_specs=[out_spec] * 4,
      out_shape=[packed] * 4,
  )(t_sg, t_ig, t_sm, t_im)


def _sc_gather_body(row_w, sidx, iidx, t_sg, t_ig, t_sm, t_im,
                    o_sg, o_ig, o_sm, o_im,
                    idxs, idxi, bsg, big, bsm, bim,
                    sem0, sem1, sem2, sem3):
  wid = lax.axis_index("s") * _NC + lax.axis_index("c")
  base = wid * _BPW
  for k in range(_NCH):
    pltpu.sync_copy(sidx.at[pl.ds(base + k * _CH, _CH)], idxs.at[k])
    pltpu.sync_copy(iidx.at[pl.ds(base + k * _CH, _CH)], idxi.at[k])

  for k in range(_NCH):
    h0 = pltpu.async_copy(t_sg.at[idxs.at[k]], bsg, sem0)
    h1 = pltpu.async_copy(t_ig.at[idxi.at[k]], big, sem1)
    h2 = pltpu.async_copy(t_sm.at[idxs.at[k]], bsm, sem2)
    h3 = pltpu.async_copy(t_im.at[idxi.at[k]], bim, sem3)
    h0.wait()
    h1.wait()
    h2.wait()
    h3.wait()
    dst = pl.ds(base + k * _CH, _CH)
    for buf, out in ((bsg, o_sg), (big, o_ig), (bsm, o_sm), (bim, o_im)):
      pltpu.sync_copy(buf, out.at[dst])


def _sc_gather(row_w, sidx, iidx, t_sg, t_ig, t_sm, t_im):
  mesh = plsc.VectorSubcoreMesh(core_axis_name="c", subcore_axis_name="s",
                                num_cores=_NC, num_subcores=_NS)
  rows = jax.ShapeDtypeStruct((_BATCH, row_w), jnp.float32)
  buf = pltpu.VMEM((_CH, row_w), jnp.float32)
  f = pl.kernel(
      lambda *a: _sc_gather_body(row_w, *a),
      out_type=[rows, rows, rows, rows],
      mesh=mesh,
      scratch_types=[
          pltpu.VMEM((_NCH, _CH), jnp.int32),
          pltpu.VMEM((_NCH, _CH), jnp.int32),
          buf, buf, buf, buf,
          pltpu.SemaphoreType.DMA,
          pltpu.SemaphoreType.DMA,
          pltpu.SemaphoreType.DMA,
          pltpu.SemaphoreType.DMA,
      ],
      compiler_params=pltpu.CompilerParams(use_tc_tiling_on_sc=False),
  )
  return f(sidx, iidx, t_sg, t_ig, t_sm, t_im)


_G = _BATCH // _PACK


def _tc_mlp_body(sg, ig, sm, im, w1s, w1i, b1, w2, b2, w3, b3, wog, woh,
                 bo, out, h_ref):
  dnp = (((1,), (0,)), ((), ()))
  dn = (((1,), (1,)), ((), ()))
  smv = sm[...]
  imv = im[...]
  for q in range(_PACK):
    h = lax.dot_general(smv, w1s[q], dnp, preferred_element_type=jnp.float32)
    h += lax.dot_general(imv, w1i[q], dnp, preferred_element_type=jnp.float32)
    h_ref[q * _G:(q + 1) * _G, :] = jnp.maximum(h + b1[...], 0.0)
  h = lax.dot_general(h_ref[...], w2[...], dn,
                      preferred_element_type=jnp.float32)
  h = jnp.maximum(h + b2[...], 0.0)
  h = lax.dot_general(h, w3[...], dn, preferred_element_type=jnp.float32)
  h = jnp.maximum(h + b3[...], 0.0)
  gmf = sg[...] * ig[...]
  for q in range(_PACK):
    logit = lax.dot_general(gmf, wog[q], dnp,
                            preferred_element_type=jnp.float32)
    logit += lax.dot_general(h[q * _G:(q + 1) * _G, :], woh[...], dn,
                             preferred_element_type=jnp.float32)
    out[q * _G:(q + 1) * _G, :] = jax.nn.sigmoid(logit + bo[...])


def _tc_mlp(sg, ig, sm, im, W1, b1, W2, b2, W3, b3, Wo, bo):
  w = jnp.arange(128)
  f = jnp.arange(_EMB)
  q = jnp.arange(_PACK)
  E = (w[None, :, None] == (q[:, None, None] * _EMB + f[None, None, :])
       ).astype(jnp.float32)
  w1s = jnp.einsum("qwf,of->qwo", E, W1[:, :_EMB])
  w1i = jnp.einsum("qwf,of->qwo", E, W1[:, _EMB:])
  wog = jnp.einsum("qwf,of->qwo", E, Wo[:, :_EMB])
  woh = Wo[:, _EMB:]

  def full(shape):
    return pl.BlockSpec(shape, lambda: tuple(0 for _ in shape))

  pk = lambda x: jnp.reshape(x, (_G, 128))
  out = pl.pallas_call(
      _tc_mlp_body,
      in_specs=[
          full((_G, 128)), full((_G, 128)), full((_G, 128)), full((_G, 128)),
          full((_PACK, 128, 64)), full((_PACK, 128, 64)), full((1, 64)),
          full((32, 64)), full((1, 32)),
          full((16, 32)), full((1, 16)),
          full((_PACK, 128, 1)), full((1, _EMB)), full((1, 1)),
      ],
      out_specs=full((_BATCH, 1)),
      out_shape=jax.ShapeDtypeStruct((_BATCH, 1), jnp.float32),
      scratch_shapes=[pltpu.VMEM((_BATCH, 64), jnp.float32)],
  )(pk(sg), pk(ig), pk(sm), pk(im), w1s, w1i,
    jnp.reshape(b1, (1, 64)), W2, jnp.reshape(b2, (1, 32)),
    W3, jnp.reshape(b3, (1, 16)), wog, woh, jnp.reshape(bo, (1, 1)))
  return jnp.reshape(jnp.transpose(jnp.reshape(out, (_PACK, _G))), (_BATCH,))


@jax.jit
def kernel(sess_ids, item_ids, sess_emb_gmf, item_emb_gmf, sess_emb_mlp,
           item_emb_mlp, W1, b1, W2, b2, W3, b3, Wo, bo):
  rp_sg, rp_ig, rp_sm, rp_im = _tc_repack(
      sess_emb_gmf.T, item_emb_gmf.T, sess_emb_mlp.T, item_emb_mlp.T)

  def row1(ids):
    return (jax.lax.shift_right_logical(ids, _RSH + 3) * _RBLK
            + jnp.bitwise_and(ids, _RBLK - 1))

  rows_sg, rows_ig, rows_sm, rows_im = _sc_gather(
      128, row1(sess_ids), row1(item_ids), rp_sg, rp_ig, rp_sm, rp_im)

  r = jnp.arange(_BATCH, dtype=jnp.int32) * _PACK

  def row2(ids):
    return r + jnp.bitwise_and(jax.lax.shift_right_logical(ids, _RSH),
                               _PACK - 1)

  sidx2 = row2(sess_ids)
  iidx2 = row2(item_ids)
  flat = lambda x: jnp.reshape(x, (_BATCH * _PACK, _EMB))
  sg, ig, sm, im = _sc_gather(
      _EMB, sidx2, iidx2, flat(rows_sg), flat(rows_ig),
      flat(rows_sm), flat(rows_im))

  return _tc_mlp(sg, ig, sm, im, W1, b1, W2, b2, W3, b3, Wo, bo)
